# baseline (speedup 1.0000x reference)
import functools

import jax
import jax.numpy as jnp
from jax import lax
from jax.experimental import pallas as pl
from jax.experimental.pallas import tpu as pltpu
from jax.experimental.pallas import tpu_sc as plsc

_EPS = 1e-12
_N_SLICES = 8
_B_BLK = 64


def _sc_gather(table, idx_flat, n_rows, d, chunk):
    info = plsc.get_sparse_core_info()
    nc, ns = info.num_cores, info.num_subcores
    nw = nc * ns
    rows_per_w = n_rows // nw
    n_chunks = rows_per_w // chunk
    mesh = plsc.VectorSubcoreMesh(core_axis_name="c", subcore_axis_name="s")

    @functools.partial(
        pl.kernel,
        mesh=mesh,
        out_type=jax.ShapeDtypeStruct((n_rows, d), jnp.float32),
        scratch_types=[
            pltpu.VMEM((chunk,), jnp.int32),
            pltpu.VMEM((chunk, d), jnp.float32),
            pltpu.SemaphoreType.DMA,
        ],
    )
    def gather_kernel(table_hbm, idx_hbm, out_hbm, idx_v, rows_v, sem):
        wid = lax.axis_index("s") * nc + lax.axis_index("c")
        base = wid * rows_per_w

        def body(i, carry):
            off = base + i * chunk
            pltpu.sync_copy(idx_hbm.at[pl.ds(off, chunk)], idx_v)
            pltpu.async_copy(table_hbm.at[idx_v], rows_v, sem).wait()
            pltpu.sync_copy(rows_v, out_hbm.at[pl.ds(off, chunk)])
            return carry

        lax.fori_loop(0, n_chunks, body, 0)

    return gather_kernel(table, idx_flat)


def _tc_ln_body(g_ref, tt_ref, posc_ref, delta_ref, gamma_ref,
                beta_ref, o_ref):
    x = g_ref[...]
    t = tt_ref[...].astype(jnp.float32)[..., None]
    x = x + posc_ref[...][None, :, :] + t * delta_ref[...][None, :, :]
    mean = jnp.mean(x, axis=-1, keepdims=True)
    xc = x - mean
    var = jnp.mean(xc * xc, axis=-1, keepdims=True)
    y = xc * lax.rsqrt(var + _EPS)
    o_ref[...] = y * gamma_ref[...][None, :, :] + beta_ref[...][None, :, :]


def _tc_ln_slice(out_buf, gathered_k, tt_k, posc, delta, gamma2, beta2,
                 slice_base, b, s, d):
    b_slice = gathered_k.shape[0]
    grid = (b_slice // _B_BLK,)
    blk0 = slice_base // _B_BLK
    out_spec = pl.BlockSpec((_B_BLK, s, d), lambda i: (blk0 + i, 0, 0))
    in_specs = [
        pl.BlockSpec((_B_BLK, s, d), lambda i: (i, 0, 0)),
        pl.BlockSpec((_B_BLK, s), lambda i: (i, 0)),
        pl.BlockSpec((s, d), lambda i: (0, 0)),
        pl.BlockSpec((1, d), lambda i: (0, 0)),
        pl.BlockSpec((1, d), lambda i: (0, 0)),
        pl.BlockSpec((1, d), lambda i: (0, 0)),
    ]
    args = (gathered_k, tt_k, posc, delta, gamma2, beta2)
    if out_buf is None:
        body = _tc_ln_body
        aliases = {}
    else:
        body = lambda o_in, *rest: _tc_ln_body(*rest)
        in_specs = [out_spec] + in_specs
        args = (out_buf,) + args
        aliases = {0: 0}
    return pl.pallas_call(
        body,
        grid=grid,
        in_specs=in_specs,
        out_specs=out_spec,
        out_shape=jax.ShapeDtypeStruct((b, s, d), jnp.float32),
        input_output_aliases=aliases,
    )(*args)


def kernel(input_ids, token_type_ids, word_emb, pos_emb, type_emb, gamma, beta):
    b, s = input_ids.shape
    d = word_emb.shape[1]
    posc = pos_emb[:s] + type_emb[0][None, :]
    delta = (type_emb[1] - type_emb[0])[None, :]
    gamma2 = gamma[None, :]
    beta2 = beta[None, :]
    ids32 = input_ids.astype(jnp.int32)
    tt32 = token_type_ids.astype(jnp.int32)

    b_slice = b // _N_SLICES
    gathered = [
        _sc_gather(
            word_emb,
            ids32[k * b_slice:(k + 1) * b_slice].reshape(-1),
            b_slice * s,
            d,
            chunk=400,
        ).reshape(b_slice, s, d)
        for k in range(_N_SLICES)
    ]

    out = None
    for k in range(_N_SLICES):
        out = _tc_ln_slice(
            out,
            gathered[k],
            tt32[k * b_slice:(k + 1) * b_slice],
            posc,
            delta,
            gamma2,
            beta2,
            slice_base=k * b_slice,
            b=b, s=s, d=d,
        )
    return out


# speedup vs baseline: 1.2335x; 1.2335x over previous
import functools

import jax
import jax.numpy as jnp
from jax import lax
from jax.experimental import pallas as pl
from jax.experimental.pallas import tpu as pltpu
from jax.experimental.pallas import tpu_sc as plsc

_EPS = 1e-12
_N_SLICES = 4
_B_BLK = 64


def _sc_gather(table, idx_flat, n_rows, d, chunk):
    info = plsc.get_sparse_core_info()
    nc, ns = info.num_cores, info.num_subcores
    nw = nc * ns
    rows_per_w = n_rows // nw
    n_chunks = rows_per_w // chunk
    mesh = plsc.VectorSubcoreMesh(core_axis_name="c", subcore_axis_name="s")

    @functools.partial(
        pl.kernel,
        mesh=mesh,
        out_type=jax.ShapeDtypeStruct((n_rows, d), jnp.float32),
        scratch_types=[
            pltpu.VMEM((chunk,), jnp.int32),
            pltpu.VMEM((chunk, d), jnp.float32),
            pltpu.SemaphoreType.DMA,
        ],
    )
    def gather_kernel(table_hbm, idx_hbm, out_hbm, idx_v, rows_v, sem):
        wid = lax.axis_index("s") * nc + lax.axis_index("c")
        base = wid * rows_per_w

        def body(i, carry):
            off = base + i * chunk
            pltpu.sync_copy(idx_hbm.at[pl.ds(off, chunk)], idx_v)
            pltpu.async_copy(table_hbm.at[idx_v], rows_v, sem).wait()
            pltpu.sync_copy(rows_v, out_hbm.at[pl.ds(off, chunk)])
            return carry

        lax.fori_loop(0, n_chunks, body, 0)

    return gather_kernel(table, idx_flat)


def _tc_ln_body(g_ref, tt_ref, posc_ref, delta_ref, gamma_ref,
                beta_ref, o_ref):
    x = g_ref[...]
    t = tt_ref[...].astype(jnp.float32)[..., None]
    x = x + posc_ref[...][None, :, :] + t * delta_ref[...][None, :, :]
    mean = jnp.mean(x, axis=-1, keepdims=True)
    xc = x - mean
    var = jnp.mean(xc * xc, axis=-1, keepdims=True)
    y = xc * lax.rsqrt(var + _EPS)
    o_ref[...] = y * gamma_ref[...][None, :, :] + beta_ref[...][None, :, :]


def _tc_ln_slice(out_buf, gathered_k, tt_k, posc, delta, gamma2, beta2,
                 slice_base, b, s, d):
    b_slice = gathered_k.shape[0]
    grid = (b_slice // _B_BLK,)
    blk0 = slice_base // _B_BLK
    out_spec = pl.BlockSpec((_B_BLK, s, d), lambda i: (blk0 + i, 0, 0))
    in_specs = [
        pl.BlockSpec((_B_BLK, s, d), lambda i: (i, 0, 0)),
        pl.BlockSpec((_B_BLK, s), lambda i: (i, 0)),
        pl.BlockSpec((s, d), lambda i: (0, 0)),
        pl.BlockSpec((1, d), lambda i: (0, 0)),
        pl.BlockSpec((1, d), lambda i: (0, 0)),
        pl.BlockSpec((1, d), lambda i: (0, 0)),
    ]
    args = (gathered_k, tt_k, posc, delta, gamma2, beta2)
    if out_buf is None:
        body = _tc_ln_body
        aliases = {}
    else:
        body = lambda o_in, *rest: _tc_ln_body(*rest)
        in_specs = [pl.BlockSpec((1, 8, d), lambda i: (0, 0, 0))] + in_specs
        args = (out_buf,) + args
        aliases = {0: 0}
    return pl.pallas_call(
        body,
        grid=grid,
        in_specs=in_specs,
        out_specs=out_spec,
        out_shape=jax.ShapeDtypeStruct((b, s, d), jnp.float32),
        input_output_aliases=aliases,
    )(*args)


def kernel(input_ids, token_type_ids, word_emb, pos_emb, type_emb, gamma, beta):
    b, s = input_ids.shape
    d = word_emb.shape[1]
    posc = pos_emb[:s] + type_emb[0][None, :]
    delta = (type_emb[1] - type_emb[0])[None, :]
    gamma2 = gamma[None, :]
    beta2 = beta[None, :]
    ids32 = input_ids.astype(jnp.int32)
    tt32 = token_type_ids.astype(jnp.int32)

    b_slice = b // _N_SLICES
    gathered = [
        _sc_gather(
            word_emb,
            ids32[k * b_slice:(k + 1) * b_slice].reshape(-1),
            b_slice * s,
            d,
            chunk=800,
        ).reshape(b_slice, s, d)
        for k in range(_N_SLICES)
    ]

    out = None
    for k in range(_N_SLICES):
        out = _tc_ln_slice(
            out,
            gathered[k],
            tt32[k * b_slice:(k + 1) * b_slice],
            posc,
            delta,
            gamma2,
            beta2,
            slice_base=k * b_slice,
            b=b, s=s, d=d,
        )
    return out
